# Initial kernel scaffold; baseline (speedup 1.0000x reference)
import jax
import jax.numpy as jnp
from jax.experimental import pallas as pl


def kernel(init_x, edge_index_rel0, edge_weight_rel0, edge_index_rel1, edge_weight_rel1, W_0_0, b_0_0, W_0_1, b_0_1, ln0_g, ln0_b, W_1_0, b_1_0, W_1_1, b_1_1, ln1_g, ln1_b):
    raise NotImplementedError("write your pallas kernel here")


# speedup vs baseline: 4.1740x; 4.1740x over previous
import dataclasses
import functools
import math

import jax
import jax.numpy as jnp
from jax import lax
from jax.experimental import pallas as pl
from jax.experimental.pallas import tpu as pltpu
from jax.experimental.pallas import tpu_sc as plsc

N = 10000
D = 256
DH = 128
E = 160000
L = 16
NSC = 2
NSUB = 16
NW = NSC * NSUB
NPAD = 10240
RPS = NPAD // NSUB
EPAD = 163840
EPT = EPAD // NSUB
KE = 128
NBLK = EPT // KE
EA = EPAD // NW
EPS = 1e-12
F32 = jnp.float32

_mesh = plsc.VectorSubcoreMesh(core_axis_name="core", subcore_axis_name="subcore")

_sc_params = pltpu.CompilerParams()
if "needs_layout_passes" in pltpu.CompilerParams.__dataclass_fields__:
    _sc_params = dataclasses.replace(_sc_params, needs_layout_passes=False)


def _deg_parts(dst0, ew0, dst1, ew1):
    @functools.partial(
        pl.kernel,
        out_type=(
            jax.ShapeDtypeStruct((NW, NPAD), F32),
            jax.ShapeDtypeStruct((NW, NPAD), F32),
        ),
        mesh=_mesh,
        scratch_types=[
            pltpu.VMEM((EA,), jnp.int32),
            pltpu.VMEM((EA,), F32),
            pltpu.VMEM((NPAD,), F32),
        ],
        compiler_params=_sc_params,
    )
    def kern(dst0_h, ew0_h, dst1_h, ew1_h, p0_h, p1_h, dstv, ewv, degv):
        c = lax.axis_index("core")
        s = lax.axis_index("subcore")
        w = c * NSUB + s
        for dh, eh, ph in ((dst0_h, ew0_h, p0_h), (dst1_h, ew1_h, p1_h)):
            base = w * EA
            pltpu.sync_copy(dh.at[pl.ds(base, EA)], dstv)
            pltpu.sync_copy(eh.at[pl.ds(base, EA)], ewv)

            @pl.loop(0, NPAD // L)
            def _zero(i):
                degv[pl.ds(i * L, L)] = jnp.zeros((L,), F32)

            lane = lax.iota(jnp.int32, L)

            @pl.loop(0, EA // L)
            def _acc(gi):
                dvec = dstv[pl.ds(gi * L, L)]
                evec = ewv[pl.ds(gi * L, L)]
                for k in range(L):
                    plsc.addupdate_scatter(degv, [dvec], evec, mask=lane == k)

            pltpu.sync_copy(degv, ph.at[w])

    return kern(dst0, ew0, dst1, ew1)


def _agg_rel(src, dst, ew, hlo, hhi):
    @functools.partial(
        pl.kernel,
        out_type=(
            jax.ShapeDtypeStruct((NPAD, DH), F32),
            jax.ShapeDtypeStruct((NPAD, DH), F32),
        ),
        mesh=_mesh,
        scratch_types=[
            pltpu.VMEM_SHARED((NPAD, DH), F32),
            pltpu.VMEM((KE,), jnp.int32),
            pltpu.VMEM((KE,), jnp.int32),
            pltpu.VMEM((KE,), F32),
            pltpu.VMEM((KE, DH), F32),
            pltpu.VMEM((KE, DH), F32),
            pltpu.SemaphoreType.DMA,
        ],
        compiler_params=_sc_params,
    )
    def kern(src_h, dst_h, ew_h, hlo_h, hhi_h, outlo, outhi,
             acc, sidx, didx, ewv, rows, zbuf, sem):
        c = lax.axis_index("core")
        s = lax.axis_index("subcore")

        def run(h_t, out_t):
            @pl.loop(0, KE)
            def _zb(r):
                for ch in range(DH // L):
                    zbuf[r, pl.ds(ch * L, L)] = jnp.zeros((L,), F32)

            @pl.loop(0, RPS // KE)
            def _za(j):
                pltpu.sync_copy(zbuf, acc.at[pl.ds(s * RPS + j * KE, KE)])

            plsc.subcore_barrier()

            base0 = s * EPT

            @pl.loop(0, NBLK)
            def _blk(i):
                base = base0 + i * KE
                pltpu.sync_copy(src_h.at[pl.ds(base, KE)], sidx)
                pltpu.sync_copy(dst_h.at[pl.ds(base, KE)], didx)
                pltpu.sync_copy(ew_h.at[pl.ds(base, KE)], ewv)
                pltpu.async_copy(h_t.at[sidx], rows, sem).wait()

                @pl.loop(0, KE)
                def _scale(e):
                    eidx = jnp.full((L,), 0, jnp.int32) + e
                    wv = plsc.load_gather(ewv, [eidx])
                    for ch in range(DH // L):
                        sl = pl.ds(ch * L, L)
                        rows[e, sl] = rows[e, sl] * wv

                pltpu.sync_copy(rows, acc.at[didx], add=True)

            plsc.subcore_barrier()
            pltpu.sync_copy(acc.at[pl.ds(s * RPS, RPS)],
                            out_t.at[pl.ds(s * RPS, RPS)])

        @pl.when(c == 0)
        def _c0():
            run(hlo_h, outlo)

        @pl.when(c == 1)
        def _c1():
            run(hhi_h, outhi)

    return kern(src, dst, ew, hlo, hhi)


def _dinv(p0, p1):
    CB = 1024

    def body(p0_ref, p1_ref, o0, o1):
        ones = jnp.ones((NW, 1), F32)
        dn = (((0,), (0,)), ((), ()))
        for p_ref, o in ((p0_ref, o0), (p1_ref, o1)):
            deg = lax.dot_general(p_ref[...], ones, dn,
                                  precision=lax.Precision.HIGHEST)
            o[...] = lax.rsqrt(deg + 1.0)

    return pl.pallas_call(
        body,
        grid=(NPAD // CB,),
        in_specs=[pl.BlockSpec((NW, CB), lambda i: (0, i))] * 2,
        out_specs=[pl.BlockSpec((CB, 1), lambda i: (i, 0))] * 2,
        out_shape=[jax.ShapeDtypeStruct((NPAD, 1), F32)] * 2,
    )(p0, p1)


def _matmul2(x, w0, w1, dinv0, dinv1):
    RB = 512

    def body(x_ref, w0_ref, w1_ref, d0_ref, d1_ref, o0l, o0h, o1l, o1h):
        xb = x_ref[...]
        for w_ref, d_ref, ol, oh in ((w0_ref, d0_ref, o0l, o0h),
                                     (w1_ref, d1_ref, o1l, o1h)):
            h = jnp.dot(xb, w_ref[...]) * d_ref[...]
            ol[...] = h[:, :DH]
            oh[...] = h[:, DH:]

    return pl.pallas_call(
        body,
        grid=(NPAD // RB,),
        in_specs=[
            pl.BlockSpec((RB, D), lambda i: (i, 0)),
            pl.BlockSpec((D, D), lambda i: (0, 0)),
            pl.BlockSpec((D, D), lambda i: (0, 0)),
            pl.BlockSpec((RB, 1), lambda i: (i, 0)),
            pl.BlockSpec((RB, 1), lambda i: (i, 0)),
        ],
        out_specs=[pl.BlockSpec((RB, DH), lambda i: (i, 0))] * 4,
        out_shape=[jax.ShapeDtypeStruct((NPAD, DH), F32)] * 4,
    )(x, w0, w1, dinv0, dinv1)


def _combine(a0l, a0h, a1l, a1h, h0l, h0h, h1l, h1h, dinv0, dinv1,
             b0, b1, g, bb):
    RB = 512
    inv_sqrt2 = 1.0 / math.sqrt(2.0)

    def body(a0l_r, a0h_r, a1l_r, a1h_r, h0l_r, h0h_r, h1l_r, h1h_r,
             d0_r, d1_r, b0_r, b1_r, g_r, bb_r, o_r):
        d0 = d0_r[...]
        d1 = d1_r[...]
        lo = d0 * (a0l_r[...] + h0l_r[...]) + d1 * (a1l_r[...] + h1l_r[...])
        hi = d0 * (a0h_r[...] + h0h_r[...]) + d1 * (a1h_r[...] + h1h_r[...])
        x = jnp.concatenate([lo, hi], axis=1) + b0_r[...] + b1_r[...]
        mu = jnp.mean(x, axis=1, keepdims=True)
        xc = x - mu
        var = jnp.mean(xc * xc, axis=1, keepdims=True)
        x = xc * lax.rsqrt(var + EPS) * g_r[...] + bb_r[...]
        o_r[...] = x * 0.5 * (1.0 + lax.erf(x * inv_sqrt2))

    row = lambda v: pl.BlockSpec((RB, DH), lambda i: (i, 0))
    return pl.pallas_call(
        body,
        grid=(NPAD // RB,),
        in_specs=(
            [pl.BlockSpec((RB, DH), lambda i: (i, 0))] * 8
            + [pl.BlockSpec((RB, 1), lambda i: (i, 0))] * 2
            + [pl.BlockSpec((1, D), lambda i: (0, 0))] * 4
        ),
        out_specs=pl.BlockSpec((RB, D), lambda i: (i, 0)),
        out_shape=jax.ShapeDtypeStruct((NPAD, D), F32),
    )(a0l, a0h, a1l, a1h, h0l, h0h, h1l, h1h, dinv0, dinv1, b0, b1, g, bb)


def kernel(init_x, edge_index_rel0, edge_weight_rel0, edge_index_rel1,
           edge_weight_rel1, W_0_0, b_0_0, W_0_1, b_0_1, ln0_g, ln0_b,
           W_1_0, b_1_0, W_1_1, b_1_1, ln1_g, ln1_b):
    x = jnp.pad(init_x.astype(F32), ((0, NPAD - N), (0, 0)))

    def prep(ei, ew):
        src = ei[0].astype(jnp.int32)
        dst = ei[1].astype(jnp.int32)
        ew = ew.astype(F32)
        pe = EPAD - E
        srcp = jnp.concatenate([src, jnp.zeros((pe,), jnp.int32)])
        dstp = jnp.concatenate([dst, jnp.zeros((pe,), jnp.int32)])
        ewp = jnp.concatenate([ew, jnp.zeros((pe,), F32)])
        return src, dst, ew, srcp, dstp, ewp

    src0, dst0, w0e, src0p, dst0p, ew0p = prep(edge_index_rel0, edge_weight_rel0)
    src1, dst1, w1e, src1p, dst1p, ew1p = prep(edge_index_rel1, edge_weight_rel1)

    p0, p1 = _deg_parts(dst0p, ew0p, dst1p, ew1p)
    dinv0, dinv1 = _dinv(p0, p1)

    layers = (
        (W_0_0, b_0_0, W_0_1, b_0_1, ln0_g, ln0_b),
        (W_1_0, b_1_0, W_1_1, b_1_1, ln1_g, ln1_b),
    )
    last = x
    for Wa, ba, Wb, bcur, g, lb in layers:
        h0l, h0h, h1l, h1h = _matmul2(last, Wa, Wb, dinv0, dinv1)
        a0l, a0h = _agg_rel(src0p, dst0p, ew0p, h0l, h0h)
        a1l, a1h = _agg_rel(src1p, dst1p, ew1p, h1l, h1h)
        last = _combine(a0l, a0h, a1l, a1h, h0l, h0h, h1l, h1h, dinv0, dinv1,
                        ba.reshape(1, D), bcur.reshape(1, D),
                        g.reshape(1, D), lb.reshape(1, D))
    return last[:N]


# speedup vs baseline: 6.9123x; 1.6561x over previous
import dataclasses
import functools
import math

import jax
import jax.numpy as jnp
from jax import lax
from jax.experimental import pallas as pl
from jax.experimental.pallas import tpu as pltpu
from jax.experimental.pallas import tpu_sc as plsc

N = 10000
D = 256
DH = 128
E = 160000
L = 16
NSC = 2
NSUB = 16
NW = NSC * NSUB
NPAD = 10240
RPS = NPAD // NSUB
EPAD = 163840
EPT = EPAD // NSUB
KE = 64
NBLK = EPT // KE
NR = 4
NI = 8
EA = EPAD // NW
EPS = 1e-12
F32 = jnp.float32

_mesh = plsc.VectorSubcoreMesh(core_axis_name="core", subcore_axis_name="subcore")

_sc_params = pltpu.CompilerParams()
if "needs_layout_passes" in pltpu.CompilerParams.__dataclass_fields__:
    _sc_params = dataclasses.replace(_sc_params, needs_layout_passes=False)


def _deg_parts(dst0, ew0, dst1, ew1):
    @functools.partial(
        pl.kernel,
        out_type=(
            jax.ShapeDtypeStruct((NW, NPAD), F32),
            jax.ShapeDtypeStruct((NW, NPAD), F32),
        ),
        mesh=_mesh,
        scratch_types=[
            pltpu.VMEM((EA,), jnp.int32),
            pltpu.VMEM((EA,), F32),
            pltpu.VMEM((NPAD,), F32),
        ],
        compiler_params=_sc_params,
    )
    def kern(dst0_h, ew0_h, dst1_h, ew1_h, p0_h, p1_h, dstv, ewv, degv):
        c = lax.axis_index("core")
        s = lax.axis_index("subcore")
        w = c * NSUB + s
        for dh, eh, ph in ((dst0_h, ew0_h, p0_h), (dst1_h, ew1_h, p1_h)):
            base = w * EA
            pltpu.sync_copy(dh.at[pl.ds(base, EA)], dstv)
            pltpu.sync_copy(eh.at[pl.ds(base, EA)], ewv)

            @pl.loop(0, NPAD // L)
            def _zero(i):
                degv[pl.ds(i * L, L)] = jnp.zeros((L,), F32)

            lane = lax.iota(jnp.int32, L)

            @pl.loop(0, EA // L)
            def _acc(gi):
                dvec = dstv[pl.ds(gi * L, L)]
                evec = ewv[pl.ds(gi * L, L)]
                for k in range(L):
                    plsc.addupdate_scatter(degv, [dvec], evec, mask=lane == k)

            pltpu.sync_copy(degv, ph.at[w])

    return kern(dst0, ew0, dst1, ew1)


def _agg_rel(src, dst, ew, hlo, hhi):
    @functools.partial(
        pl.kernel,
        out_type=(
            jax.ShapeDtypeStruct((NPAD, DH), F32),
            jax.ShapeDtypeStruct((NPAD, DH), F32),
        ),
        mesh=_mesh,
        scratch_types=[
            pltpu.VMEM_SHARED((NPAD, DH), F32),
            pltpu.VMEM((NI, KE), jnp.int32),
            pltpu.VMEM((NI, KE), jnp.int32),
            pltpu.VMEM((NI, KE), F32),
        ]
        + [pltpu.VMEM((KE, DH), F32)] * NR
        + [pltpu.SemaphoreType.DMA] * (2 * NR + NI),
        compiler_params=_sc_params,
    )
    def kern(src_h, dst_h, ew_h, hlo_h, hhi_h, outlo, outhi,
             acc, sidx, didx, eww, *bufsem):
        c = lax.axis_index("core")
        s = lax.axis_index("subcore")
        rows = bufsem[:NR]
        gsem = bufsem[NR:2 * NR]
        ssem = bufsem[2 * NR:3 * NR]
        isem = bufsem[3 * NR:]

        def run(h_t, out_t):
            base0 = s * EPT

            def idx_issue(j, slot):
                base = base0 + j * KE
                pltpu.async_copy(src_h.at[pl.ds(base, KE)], sidx.at[slot],
                                 isem[slot])
                pltpu.async_copy(dst_h.at[pl.ds(base, KE)], didx.at[slot],
                                 isem[slot])
                pltpu.async_copy(ew_h.at[pl.ds(base, KE)], eww.at[slot],
                                 isem[slot])

            def idx_wait(j, slot):
                base = base0 + j * KE
                pltpu.make_async_copy(src_h.at[pl.ds(base, KE)],
                                      sidx.at[slot], isem[slot]).wait()
                pltpu.make_async_copy(dst_h.at[pl.ds(base, KE)],
                                      didx.at[slot], isem[slot]).wait()
                pltpu.make_async_copy(ew_h.at[pl.ds(base, KE)],
                                      eww.at[slot], isem[slot]).wait()

            r0 = rows[0]

            @pl.loop(0, KE)
            def _zb(r):
                for ch in range(DH // L):
                    r0[r, pl.ds(ch * L, L)] = jnp.zeros((L,), F32)

            @pl.loop(0, RPS // KE)
            def _za(j):
                pltpu.sync_copy(r0, acc.at[pl.ds(s * RPS + j * KE, KE)])

            plsc.subcore_barrier()

            def scale(slot, rbuf):
                @pl.loop(0, KE)
                def _sc(e):
                    eidx = jnp.full((L,), 0, jnp.int32) + e
                    gv = jnp.full((L,), 0, jnp.int32) + slot
                    wv = plsc.load_gather(eww, [gv, eidx])
                    for ch in range(DH // L):
                        sl = pl.ds(ch * L, L)
                        rbuf[e, sl] = rbuf[e, sl] * wv

            for j in range(3):
                idx_issue(j, j)
            idx_wait(0, 0)
            pltpu.async_copy(h_t.at[sidx.at[0]], rows[0], gsem[0])
            idx_wait(1, 1)
            pltpu.async_copy(h_t.at[sidx.at[1]], rows[1], gsem[1])

            @pl.loop(0, NBLK, step=NI)
            def _outer(go):
                for b in range(NI):
                    g = go + b
                    br = b % NR
                    rb = rows[br]
                    b2 = (b + 2) % NR

                    @pl.when(g + 3 < NBLK)
                    def _iss():
                        idx_issue(g + 3, (b + 3) % NI)

                    pltpu.make_async_copy(h_t.at[sidx.at[b]], rb,
                                          gsem[br]).wait()
                    scale(b, rb)
                    pltpu.async_copy(rb, acc.at[didx.at[b]], ssem[br],
                                     add=True)

                    @pl.when(g + 2 < NBLK)
                    def _pref():
                        @pl.when(g >= 2)
                        def _w():
                            pltpu.make_async_copy(
                                rows[b2], acc.at[didx.at[(b - 2) % NI]],
                                ssem[b2]).wait()

                        idx_wait(g + 2, (b + 2) % NI)
                        pltpu.async_copy(h_t.at[sidx.at[(b + 2) % NI]],
                                         rows[b2], gsem[b2])

            for j in range(NR):
                g = NBLK - NR + j
                pltpu.make_async_copy(rows[g % NR], acc.at[didx.at[g % NI]],
                                      ssem[g % NR]).wait()

            plsc.subcore_barrier()
            pltpu.sync_copy(acc.at[pl.ds(s * RPS, RPS)],
                            out_t.at[pl.ds(s * RPS, RPS)])

        @pl.when(c == 0)
        def _c0():
            run(hlo_h, outlo)

        @pl.when(c == 1)
        def _c1():
            run(hhi_h, outhi)

    return kern(src, dst, ew, hlo, hhi)


def _dinv(p0, p1):
    CB = 1024

    def body(p0_ref, p1_ref, o0, o1):
        ones = jnp.ones((NW, 1), F32)
        dn = (((0,), (0,)), ((), ()))
        for p_ref, o in ((p0_ref, o0), (p1_ref, o1)):
            deg = lax.dot_general(p_ref[...], ones, dn,
                                  precision=lax.Precision.HIGHEST)
            o[...] = lax.rsqrt(deg + 1.0)

    return pl.pallas_call(
        body,
        grid=(NPAD // CB,),
        in_specs=[pl.BlockSpec((NW, CB), lambda i: (0, i))] * 2,
        out_specs=[pl.BlockSpec((CB, 1), lambda i: (i, 0))] * 2,
        out_shape=[jax.ShapeDtypeStruct((NPAD, 1), F32)] * 2,
    )(p0, p1)


def _matmul2(x, w0, w1, dinv0, dinv1):
    RB = 512

    def body(x_ref, w0_ref, w1_ref, d0_ref, d1_ref, o0l, o0h, o1l, o1h):
        xb = x_ref[...]
        for w_ref, d_ref, ol, oh in ((w0_ref, d0_ref, o0l, o0h),
                                     (w1_ref, d1_ref, o1l, o1h)):
            h = jnp.dot(xb, w_ref[...]) * d_ref[...]
            ol[...] = h[:, :DH]
            oh[...] = h[:, DH:]

    return pl.pallas_call(
        body,
        grid=(NPAD // RB,),
        in_specs=[
            pl.BlockSpec((RB, D), lambda i: (i, 0)),
            pl.BlockSpec((D, D), lambda i: (0, 0)),
            pl.BlockSpec((D, D), lambda i: (0, 0)),
            pl.BlockSpec((RB, 1), lambda i: (i, 0)),
            pl.BlockSpec((RB, 1), lambda i: (i, 0)),
        ],
        out_specs=[pl.BlockSpec((RB, DH), lambda i: (i, 0))] * 4,
        out_shape=[jax.ShapeDtypeStruct((NPAD, DH), F32)] * 4,
    )(x, w0, w1, dinv0, dinv1)


def _combine(a0l, a0h, a1l, a1h, h0l, h0h, h1l, h1h, dinv0, dinv1,
             b0, b1, g, bb):
    RB = 512
    inv_sqrt2 = 1.0 / math.sqrt(2.0)

    def body(a0l_r, a0h_r, a1l_r, a1h_r, h0l_r, h0h_r, h1l_r, h1h_r,
             d0_r, d1_r, b0_r, b1_r, g_r, bb_r, o_r):
        d0 = d0_r[...]
        d1 = d1_r[...]
        lo = d0 * (a0l_r[...] + h0l_r[...]) + d1 * (a1l_r[...] + h1l_r[...])
        hi = d0 * (a0h_r[...] + h0h_r[...]) + d1 * (a1h_r[...] + h1h_r[...])
        x = jnp.concatenate([lo, hi], axis=1) + b0_r[...] + b1_r[...]
        mu = jnp.mean(x, axis=1, keepdims=True)
        xc = x - mu
        var = jnp.mean(xc * xc, axis=1, keepdims=True)
        x = xc * lax.rsqrt(var + EPS) * g_r[...] + bb_r[...]
        o_r[...] = x * 0.5 * (1.0 + lax.erf(x * inv_sqrt2))

    row = lambda v: pl.BlockSpec((RB, DH), lambda i: (i, 0))
    return pl.pallas_call(
        body,
        grid=(NPAD // RB,),
        in_specs=(
            [pl.BlockSpec((RB, DH), lambda i: (i, 0))] * 8
            + [pl.BlockSpec((RB, 1), lambda i: (i, 0))] * 2
            + [pl.BlockSpec((1, D), lambda i: (0, 0))] * 4
        ),
        out_specs=pl.BlockSpec((RB, D), lambda i: (i, 0)),
        out_shape=jax.ShapeDtypeStruct((NPAD, D), F32),
    )(a0l, a0h, a1l, a1h, h0l, h0h, h1l, h1h, dinv0, dinv1, b0, b1, g, bb)


def kernel(init_x, edge_index_rel0, edge_weight_rel0, edge_index_rel1,
           edge_weight_rel1, W_0_0, b_0_0, W_0_1, b_0_1, ln0_g, ln0_b,
           W_1_0, b_1_0, W_1_1, b_1_1, ln1_g, ln1_b):
    x = jnp.pad(init_x.astype(F32), ((0, NPAD - N), (0, 0)))

    def prep(ei, ew):
        src = ei[0].astype(jnp.int32)
        dst = ei[1].astype(jnp.int32)
        ew = ew.astype(F32)
        pe = EPAD - E
        srcp = jnp.concatenate([src, jnp.zeros((pe,), jnp.int32)])
        dstp = jnp.concatenate([dst, jnp.zeros((pe,), jnp.int32)])
        ewp = jnp.concatenate([ew, jnp.zeros((pe,), F32)])
        return srcp, dstp, ewp

    src0p, dst0p, ew0p = prep(edge_index_rel0, edge_weight_rel0)
    src1p, dst1p, ew1p = prep(edge_index_rel1, edge_weight_rel1)

    p0, p1 = _deg_parts(dst0p, ew0p, dst1p, ew1p)
    dinv0, dinv1 = _dinv(p0, p1)

    layers = (
        (W_0_0, b_0_0, W_0_1, b_0_1, ln0_g, ln0_b),
        (W_1_0, b_1_0, W_1_1, b_1_1, ln1_g, ln1_b),
    )
    last = x
    for Wa, ba, Wb, bcur, g, lb in layers:
        h0l, h0h, h1l, h1h = _matmul2(last, Wa, Wb, dinv0, dinv1)
        a0l, a0h = _agg_rel(src0p, dst0p, ew0p, h0l, h0h)
        a1l, a1h = _agg_rel(src1p, dst1p, ew1p, h1l, h1h)
        last = _combine(a0l, a0h, a1l, a1h, h0l, h0h, h1l, h1h, dinv0, dinv1,
                        ba.reshape(1, D), bcur.reshape(1, D),
                        g.reshape(1, D), lb.reshape(1, D))
    return last[:N]


# speedup vs baseline: 7.0333x; 1.0175x over previous
import dataclasses
import functools
import math

import jax
import jax.numpy as jnp
from jax import lax
from jax.experimental import pallas as pl
from jax.experimental.pallas import tpu as pltpu
from jax.experimental.pallas import tpu_sc as plsc

N = 10000
D = 256
DH = 128
E = 160000
L = 16
NSC = 2
NSUB = 16
NW = NSC * NSUB
NPAD = 10240
RPS = NPAD // NSUB
EPAD = 163840
EPT = EPAD // NSUB
KE = 64
NBLK = EPT // KE
NR = 4
NI = 8
EA = EPAD // NW
EPS = 1e-12
F32 = jnp.float32

_mesh = plsc.VectorSubcoreMesh(core_axis_name="core", subcore_axis_name="subcore")

_sc_params = pltpu.CompilerParams()
if "needs_layout_passes" in pltpu.CompilerParams.__dataclass_fields__:
    _sc_params = dataclasses.replace(_sc_params, needs_layout_passes=False)


def _deg_parts(dst0, ew0, dst1, ew1):
    @functools.partial(
        pl.kernel,
        out_type=(
            jax.ShapeDtypeStruct((NW, NPAD), F32),
            jax.ShapeDtypeStruct((NW, NPAD), F32),
        ),
        mesh=_mesh,
        scratch_types=[
            pltpu.VMEM((EA,), jnp.int32),
            pltpu.VMEM((EA,), F32),
            pltpu.VMEM((NPAD,), F32),
        ],
        compiler_params=_sc_params,
    )
    def kern(dst0_h, ew0_h, dst1_h, ew1_h, p0_h, p1_h, dstv, ewv, degv):
        c = lax.axis_index("core")
        s = lax.axis_index("subcore")
        w = c * NSUB + s
        for dh, eh, ph in ((dst0_h, ew0_h, p0_h), (dst1_h, ew1_h, p1_h)):
            base = w * EA
            pltpu.sync_copy(dh.at[pl.ds(base, EA)], dstv)
            pltpu.sync_copy(eh.at[pl.ds(base, EA)], ewv)

            @pl.loop(0, NPAD // L)
            def _zero(i):
                degv[pl.ds(i * L, L)] = jnp.zeros((L,), F32)

            lane = lax.iota(jnp.int32, L)

            @pl.loop(0, EA // L)
            def _acc(gi):
                dvec = dstv[pl.ds(gi * L, L)]
                evec = ewv[pl.ds(gi * L, L)]
                for k in range(L):
                    plsc.addupdate_scatter(degv, [dvec], evec, mask=lane == k)

            pltpu.sync_copy(degv, ph.at[w])

    return kern(dst0, ew0, dst1, ew1)


def _agg_rel(src, dst, ew, hlo, hhi):
    @functools.partial(
        pl.kernel,
        out_type=(
            jax.ShapeDtypeStruct((NPAD, DH), F32),
            jax.ShapeDtypeStruct((NPAD, DH), F32),
        ),
        mesh=_mesh,
        scratch_types=[
            pltpu.VMEM_SHARED((NPAD, DH), F32),
            pltpu.VMEM((NI, KE), jnp.int32),
            pltpu.VMEM((NI, KE), jnp.int32),
            pltpu.VMEM((NI, KE), F32),
        ]
        + [pltpu.VMEM((KE, DH), F32)] * NR
        + [pltpu.SemaphoreType.DMA] * (2 * NR + NI),
        compiler_params=_sc_params,
    )
    def kern(src_h, dst_h, ew_h, hlo_h, hhi_h, outlo, outhi,
             acc, sidx, didx, eww, *bufsem):
        c = lax.axis_index("core")
        s = lax.axis_index("subcore")
        rows = bufsem[:NR]
        gsem = bufsem[NR:2 * NR]
        ssem = bufsem[2 * NR:3 * NR]
        isem = bufsem[3 * NR:]

        def run(h_t, out_t):
            base0 = s * EPT

            def idx_issue(j, slot):
                base = base0 + j * KE
                pltpu.async_copy(src_h.at[pl.ds(base, KE)], sidx.at[slot],
                                 isem[slot])
                pltpu.async_copy(dst_h.at[pl.ds(base, KE)], didx.at[slot],
                                 isem[slot])
                pltpu.async_copy(ew_h.at[pl.ds(base, KE)], eww.at[slot],
                                 isem[slot])

            def idx_wait(j, slot):
                base = base0 + j * KE
                pltpu.make_async_copy(src_h.at[pl.ds(base, KE)],
                                      sidx.at[slot], isem[slot]).wait()
                pltpu.make_async_copy(dst_h.at[pl.ds(base, KE)],
                                      didx.at[slot], isem[slot]).wait()
                pltpu.make_async_copy(ew_h.at[pl.ds(base, KE)],
                                      eww.at[slot], isem[slot]).wait()

            r0 = rows[0]

            @pl.loop(0, KE)
            def _zb(r):
                for ch in range(DH // L):
                    r0[r, pl.ds(ch * L, L)] = jnp.zeros((L,), F32)

            @pl.loop(0, RPS // KE)
            def _za(j):
                pltpu.sync_copy(r0, acc.at[pl.ds(s * RPS + j * KE, KE)])

            plsc.subcore_barrier()

            def scale(slot, rbuf):
                @plsc.parallel_loop(0, KE, unroll=8)
                def _sc(e):
                    eidx = jnp.full((L,), 0, jnp.int32) + e
                    gv = jnp.full((L,), slot, jnp.int32)
                    wv = plsc.load_gather(eww, [gv, eidx])
                    for ch in range(DH // L):
                        sl = pl.ds(ch * L, L)
                        rbuf[e, sl] = rbuf[e, sl] * wv

            for j in range(3):
                idx_issue(j, j)
            idx_wait(0, 0)
            pltpu.async_copy(h_t.at[sidx.at[0]], rows[0], gsem[0])
            idx_wait(1, 1)
            pltpu.async_copy(h_t.at[sidx.at[1]], rows[1], gsem[1])

            @pl.loop(0, NBLK, step=NI)
            def _outer(go):
                for b in range(NI):
                    g = go + b
                    br = b % NR
                    rb = rows[br]
                    b2 = (b + 2) % NR

                    @pl.when(g + 3 < NBLK)
                    def _iss():
                        idx_issue(g + 3, (b + 3) % NI)

                    pltpu.make_async_copy(h_t.at[sidx.at[b]], rb,
                                          gsem[br]).wait()
                    scale(b, rb)
                    pltpu.async_copy(rb, acc.at[didx.at[b]], ssem[br],
                                     add=True)

                    @pl.when(g + 2 < NBLK)
                    def _pref():
                        @pl.when(g >= 2)
                        def _w():
                            pltpu.make_async_copy(
                                rows[b2], acc.at[didx.at[(b - 2) % NI]],
                                ssem[b2]).wait()

                        idx_wait(g + 2, (b + 2) % NI)
                        pltpu.async_copy(h_t.at[sidx.at[(b + 2) % NI]],
                                         rows[b2], gsem[b2])

            for j in range(NR):
                g = NBLK - NR + j
                pltpu.make_async_copy(rows[g % NR], acc.at[didx.at[g % NI]],
                                      ssem[g % NR]).wait()

            plsc.subcore_barrier()
            pltpu.sync_copy(acc.at[pl.ds(s * RPS, RPS)],
                            out_t.at[pl.ds(s * RPS, RPS)])

        @pl.when(c == 0)
        def _c0():
            run(hlo_h, outlo)

        @pl.when(c == 1)
        def _c1():
            run(hhi_h, outhi)

    return kern(src, dst, ew, hlo, hhi)


def _dinv(p0, p1):
    CB = 1024

    def body(p0_ref, p1_ref, o0, o1):
        ones = jnp.ones((NW, 1), F32)
        dn = (((0,), (0,)), ((), ()))
        for p_ref, o in ((p0_ref, o0), (p1_ref, o1)):
            deg = lax.dot_general(p_ref[...], ones, dn,
                                  precision=lax.Precision.HIGHEST)
            o[...] = lax.rsqrt(deg + 1.0)

    return pl.pallas_call(
        body,
        grid=(NPAD // CB,),
        in_specs=[pl.BlockSpec((NW, CB), lambda i: (0, i))] * 2,
        out_specs=[pl.BlockSpec((CB, 1), lambda i: (i, 0))] * 2,
        out_shape=[jax.ShapeDtypeStruct((NPAD, 1), F32)] * 2,
    )(p0, p1)


def _matmul2(x, w0, w1, dinv0, dinv1):
    RB = 512

    def body(x_ref, w0_ref, w1_ref, d0_ref, d1_ref, o0l, o0h, o1l, o1h):
        xb = x_ref[...]
        for w_ref, d_ref, ol, oh in ((w0_ref, d0_ref, o0l, o0h),
                                     (w1_ref, d1_ref, o1l, o1h)):
            h = jnp.dot(xb, w_ref[...]) * d_ref[...]
            ol[...] = h[:, :DH]
            oh[...] = h[:, DH:]

    return pl.pallas_call(
        body,
        grid=(NPAD // RB,),
        in_specs=[
            pl.BlockSpec((RB, D), lambda i: (i, 0)),
            pl.BlockSpec((D, D), lambda i: (0, 0)),
            pl.BlockSpec((D, D), lambda i: (0, 0)),
            pl.BlockSpec((RB, 1), lambda i: (i, 0)),
            pl.BlockSpec((RB, 1), lambda i: (i, 0)),
        ],
        out_specs=[pl.BlockSpec((RB, DH), lambda i: (i, 0))] * 4,
        out_shape=[jax.ShapeDtypeStruct((NPAD, DH), F32)] * 4,
    )(x, w0, w1, dinv0, dinv1)


def _combine(a0l, a0h, a1l, a1h, h0l, h0h, h1l, h1h, dinv0, dinv1,
             b0, b1, g, bb):
    RB = 512
    inv_sqrt2 = 1.0 / math.sqrt(2.0)

    def body(a0l_r, a0h_r, a1l_r, a1h_r, h0l_r, h0h_r, h1l_r, h1h_r,
             d0_r, d1_r, b0_r, b1_r, g_r, bb_r, o_r):
        d0 = d0_r[...]
        d1 = d1_r[...]
        lo = d0 * (a0l_r[...] + h0l_r[...]) + d1 * (a1l_r[...] + h1l_r[...])
        hi = d0 * (a0h_r[...] + h0h_r[...]) + d1 * (a1h_r[...] + h1h_r[...])
        x = jnp.concatenate([lo, hi], axis=1) + b0_r[...] + b1_r[...]
        mu = jnp.mean(x, axis=1, keepdims=True)
        xc = x - mu
        var = jnp.mean(xc * xc, axis=1, keepdims=True)
        x = xc * lax.rsqrt(var + EPS) * g_r[...] + bb_r[...]
        o_r[...] = x * 0.5 * (1.0 + lax.erf(x * inv_sqrt2))

    row = lambda v: pl.BlockSpec((RB, DH), lambda i: (i, 0))
    return pl.pallas_call(
        body,
        grid=(NPAD // RB,),
        in_specs=(
            [pl.BlockSpec((RB, DH), lambda i: (i, 0))] * 8
            + [pl.BlockSpec((RB, 1), lambda i: (i, 0))] * 2
            + [pl.BlockSpec((1, D), lambda i: (0, 0))] * 4
        ),
        out_specs=pl.BlockSpec((RB, D), lambda i: (i, 0)),
        out_shape=jax.ShapeDtypeStruct((NPAD, D), F32),
    )(a0l, a0h, a1l, a1h, h0l, h0h, h1l, h1h, dinv0, dinv1, b0, b1, g, bb)


def kernel(init_x, edge_index_rel0, edge_weight_rel0, edge_index_rel1,
           edge_weight_rel1, W_0_0, b_0_0, W_0_1, b_0_1, ln0_g, ln0_b,
           W_1_0, b_1_0, W_1_1, b_1_1, ln1_g, ln1_b):
    x = jnp.pad(init_x.astype(F32), ((0, NPAD - N), (0, 0)))

    def prep(ei, ew):
        src = ei[0].astype(jnp.int32)
        dst = ei[1].astype(jnp.int32)
        ew = ew.astype(F32)
        pe = EPAD - E
        srcp = jnp.concatenate([src, jnp.zeros((pe,), jnp.int32)])
        dstp = jnp.concatenate([dst, jnp.zeros((pe,), jnp.int32)])
        ewp = jnp.concatenate([ew, jnp.zeros((pe,), F32)])
        return srcp, dstp, ewp

    src0p, dst0p, ew0p = prep(edge_index_rel0, edge_weight_rel0)
    src1p, dst1p, ew1p = prep(edge_index_rel1, edge_weight_rel1)

    p0, p1 = _deg_parts(dst0p, ew0p, dst1p, ew1p)
    dinv0, dinv1 = _dinv(p0, p1)

    layers = (
        (W_0_0, b_0_0, W_0_1, b_0_1, ln0_g, ln0_b),
        (W_1_0, b_1_0, W_1_1, b_1_1, ln1_g, ln1_b),
    )
    last = x
    for Wa, ba, Wb, bcur, g, lb in layers:
        h0l, h0h, h1l, h1h = _matmul2(last, Wa, Wb, dinv0, dinv1)
        a0l, a0h = _agg_rel(src0p, dst0p, ew0p, h0l, h0h)
        a1l, a1h = _agg_rel(src1p, dst1p, ew1p, h1l, h1h)
        last = _combine(a0l, a0h, a1l, a1h, h0l, h0h, h1l, h1h, dinv0, dinv1,
                        ba.reshape(1, D), bcur.reshape(1, D),
                        g.reshape(1, D), lb.reshape(1, D))
    return last[:N]


# speedup vs baseline: 7.3409x; 1.0437x over previous
import dataclasses
import functools
import math

import jax
import jax.numpy as jnp
from jax import lax
from jax.experimental import pallas as pl
from jax.experimental.pallas import tpu as pltpu
from jax.experimental.pallas import tpu_sc as plsc

N = 10000
D = 256
DH = 128
E = 160000
L = 16
NSC = 2
NSUB = 16
NW = NSC * NSUB
NPAD = 10240
RPS = NPAD // NSUB
EPAD = 163840
EPT = EPAD // NSUB
KE = 64
NBLK = EPT // KE
NR = 4
GB = 8
NGRP = NBLK // GB
EA = EPAD // NW
EPS = 1e-12
F32 = jnp.float32

_mesh = plsc.VectorSubcoreMesh(core_axis_name="core", subcore_axis_name="subcore")

_sc_params = pltpu.CompilerParams()
if "needs_layout_passes" in pltpu.CompilerParams.__dataclass_fields__:
    _sc_params = dataclasses.replace(_sc_params, needs_layout_passes=False)


def _deg_parts(dst0, ew0, dst1, ew1):
    @functools.partial(
        pl.kernel,
        out_type=(
            jax.ShapeDtypeStruct((NW, NPAD), F32),
            jax.ShapeDtypeStruct((NW, NPAD), F32),
        ),
        mesh=_mesh,
        scratch_types=[
            pltpu.VMEM((EA,), jnp.int32),
            pltpu.VMEM((EA,), F32),
            pltpu.VMEM((NPAD,), F32),
        ],
        compiler_params=_sc_params,
    )
    def kern(dst0_h, ew0_h, dst1_h, ew1_h, p0_h, p1_h, dstv, ewv, degv):
        c = lax.axis_index("core")
        s = lax.axis_index("subcore")
        w = c * NSUB + s
        for dh, eh, ph in ((dst0_h, ew0_h, p0_h), (dst1_h, ew1_h, p1_h)):
            base = w * EA
            pltpu.sync_copy(dh.at[pl.ds(base, EA)], dstv)
            pltpu.sync_copy(eh.at[pl.ds(base, EA)], ewv)

            @pl.loop(0, NPAD // L)
            def _zero(i):
                degv[pl.ds(i * L, L)] = jnp.zeros((L,), F32)

            lane = lax.iota(jnp.int32, L)

            @pl.loop(0, EA // L)
            def _acc(gi):
                dvec = dstv[pl.ds(gi * L, L)]
                evec = ewv[pl.ds(gi * L, L)]
                for k in range(L):
                    plsc.addupdate_scatter(degv, [dvec], evec, mask=lane == k)

            pltpu.sync_copy(degv, ph.at[w])

    return kern(dst0, ew0, dst1, ew1)


def _agg_rel(edata, hlo, hhi):
    @functools.partial(
        pl.kernel,
        out_type=(
            jax.ShapeDtypeStruct((NPAD, DH), F32),
            jax.ShapeDtypeStruct((NPAD, DH), F32),
        ),
        mesh=_mesh,
        scratch_types=[
            pltpu.VMEM_SHARED((NPAD, DH), F32),
            pltpu.VMEM((2, 3 * GB, KE), jnp.int32),
        ]
        + [pltpu.VMEM((KE, DH), F32)] * NR
        + [pltpu.SemaphoreType.DMA] * (2 * NR + 2),
        compiler_params=_sc_params,
    )
    def kern(ed_h, hlo_h, hhi_h, outlo, outhi, acc, edat, *bufsem):
        c = lax.axis_index("core")
        s = lax.axis_index("subcore")
        rows = bufsem[:NR]
        gsem = bufsem[NR:2 * NR]
        ssem = bufsem[2 * NR:3 * NR]
        esem = bufsem[3 * NR:]

        def run(h_t, out_t):
            def grp_issue(q, slot):
                pltpu.async_copy(ed_h.at[s, q], edat.at[slot], esem[slot])

            def grp_wait(q, slot):
                pltpu.make_async_copy(ed_h.at[s, q], edat.at[slot],
                                      esem[slot]).wait()

            r0 = rows[0]

            @pl.loop(0, KE)
            def _zb(r):
                for ch in range(DH // L):
                    r0[r, pl.ds(ch * L, L)] = jnp.zeros((L,), F32)

            @pl.loop(0, RPS // KE)
            def _za(j):
                pltpu.sync_copy(r0, acc.at[pl.ds(s * RPS + j * KE, KE)])

            plsc.subcore_barrier()

            def scale(slot, b, rbuf):
                @plsc.parallel_loop(0, KE, unroll=4)
                def _sc(e):
                    eidx = jnp.full((L,), 0, jnp.int32) + e
                    svec = jnp.full((L,), slot, jnp.int32)
                    rvec = jnp.full((L,), 2 * GB + b, jnp.int32)
                    wv = plsc.bitcast(
                        plsc.load_gather(edat, [svec, rvec, eidx]), F32)
                    for ch in range(DH // L):
                        sl = pl.ds(ch * L, L)
                        rbuf[e, sl] = rbuf[e, sl] * wv

            grp_issue(0, 0)
            grp_wait(0, 0)
            pltpu.async_copy(h_t.at[edat.at[0, 0]], rows[0], gsem[0])
            pltpu.async_copy(h_t.at[edat.at[0, 1]], rows[1], gsem[1])

            @pl.loop(0, NBLK, step=2 * GB)
            def _outer(go):
                qbase = go // GB
                for gb in range(2 * GB):
                    half, b = divmod(gb, GB)
                    g = go + gb
                    q1 = qbase + half + 1
                    br = gb % NR
                    b2 = (gb + 2) % NR
                    gb2 = gb + 2
                    slot2, b2r = divmod(gb2, GB)
                    slot2 %= 2

                    if b == 2:
                        @pl.when(q1 < NGRP)
                        def _gi():
                            grp_issue(q1, 1 - half)
                    if b == GB - 2:
                        @pl.when(q1 < NGRP)
                        def _gw():
                            grp_wait(q1, 1 - half)

                    pltpu.make_async_copy(h_t.at[edat.at[half, b]],
                                          rows[br], gsem[br]).wait()
                    scale(half, b, rows[br])
                    pltpu.async_copy(rows[br], acc.at[edat.at[half, GB + b]],
                                     ssem[br], add=True)

                    @pl.when(g + 2 < NBLK)
                    def _pref():
                        @pl.when(g >= 2)
                        def _w():
                            pltpu.make_async_copy(
                                rows[b2], acc.at[edat.at[half, GB + b]],
                                ssem[b2]).wait()

                        pltpu.async_copy(h_t.at[edat.at[slot2, b2r]],
                                         rows[b2], gsem[b2])

            for j in range(NR):
                g = NBLK - NR + j
                pltpu.make_async_copy(rows[g % NR],
                                      acc.at[edat.at[1, GB + (g % GB)]],
                                      ssem[g % NR]).wait()

            plsc.subcore_barrier()
            pltpu.sync_copy(acc.at[pl.ds(s * RPS, RPS)],
                            out_t.at[pl.ds(s * RPS, RPS)])

        @pl.when(c == 0)
        def _c0():
            run(hlo_h, outlo)

        @pl.when(c == 1)
        def _c1():
            run(hhi_h, outhi)

    return kern(edata, hlo, hhi)


def _dinv(p0, p1):
    CB = 1024

    def body(p0_ref, p1_ref, o0, o1):
        ones = jnp.ones((NW, 1), F32)
        dn = (((0,), (0,)), ((), ()))
        for p_ref, o in ((p0_ref, o0), (p1_ref, o1)):
            deg = lax.dot_general(p_ref[...], ones, dn,
                                  precision=lax.Precision.HIGHEST)
            o[...] = lax.rsqrt(deg + 1.0)

    return pl.pallas_call(
        body,
        grid=(NPAD // CB,),
        in_specs=[pl.BlockSpec((NW, CB), lambda i: (0, i))] * 2,
        out_specs=[pl.BlockSpec((CB, 1), lambda i: (i, 0))] * 2,
        out_shape=[jax.ShapeDtypeStruct((NPAD, 1), F32)] * 2,
    )(p0, p1)


def _matmul2(x, w0, w1, dinv0, dinv1):
    RB = 512

    def body(x_ref, w0_ref, w1_ref, d0_ref, d1_ref, o0l, o0h, o1l, o1h):
        xb = x_ref[...]
        for w_ref, d_ref, ol, oh in ((w0_ref, d0_ref, o0l, o0h),
                                     (w1_ref, d1_ref, o1l, o1h)):
            h = jnp.dot(xb, w_ref[...]) * d_ref[...]
            ol[...] = h[:, :DH]
            oh[...] = h[:, DH:]

    return pl.pallas_call(
        body,
        grid=(NPAD // RB,),
        in_specs=[
            pl.BlockSpec((RB, D), lambda i: (i, 0)),
            pl.BlockSpec((D, D), lambda i: (0, 0)),
            pl.BlockSpec((D, D), lambda i: (0, 0)),
            pl.BlockSpec((RB, 1), lambda i: (i, 0)),
            pl.BlockSpec((RB, 1), lambda i: (i, 0)),
        ],
        out_specs=[pl.BlockSpec((RB, DH), lambda i: (i, 0))] * 4,
        out_shape=[jax.ShapeDtypeStruct((NPAD, DH), F32)] * 4,
    )(x, w0, w1, dinv0, dinv1)


def _combine(a0l, a0h, a1l, a1h, h0l, h0h, h1l, h1h, dinv0, dinv1,
             b0, b1, g, bb):
    RB = 512
    inv_sqrt2 = 1.0 / math.sqrt(2.0)

    def body(a0l_r, a0h_r, a1l_r, a1h_r, h0l_r, h0h_r, h1l_r, h1h_r,
             d0_r, d1_r, b0_r, b1_r, g_r, bb_r, o_r):
        d0 = d0_r[...]
        d1 = d1_r[...]
        lo = d0 * (a0l_r[...] + h0l_r[...]) + d1 * (a1l_r[...] + h1l_r[...])
        hi = d0 * (a0h_r[...] + h0h_r[...]) + d1 * (a1h_r[...] + h1h_r[...])
        x = jnp.concatenate([lo, hi], axis=1) + b0_r[...] + b1_r[...]
        mu = jnp.mean(x, axis=1, keepdims=True)
        xc = x - mu
        var = jnp.mean(xc * xc, axis=1, keepdims=True)
        x = xc * lax.rsqrt(var + EPS) * g_r[...] + bb_r[...]
        o_r[...] = x * 0.5 * (1.0 + lax.erf(x * inv_sqrt2))

    row = lambda v: pl.BlockSpec((RB, DH), lambda i: (i, 0))
    return pl.pallas_call(
        body,
        grid=(NPAD // RB,),
        in_specs=(
            [pl.BlockSpec((RB, DH), lambda i: (i, 0))] * 8
            + [pl.BlockSpec((RB, 1), lambda i: (i, 0))] * 2
            + [pl.BlockSpec((1, D), lambda i: (0, 0))] * 4
        ),
        out_specs=pl.BlockSpec((RB, D), lambda i: (i, 0)),
        out_shape=jax.ShapeDtypeStruct((NPAD, D), F32),
    )(a0l, a0h, a1l, a1h, h0l, h0h, h1l, h1h, dinv0, dinv1, b0, b1, g, bb)


def kernel(init_x, edge_index_rel0, edge_weight_rel0, edge_index_rel1,
           edge_weight_rel1, W_0_0, b_0_0, W_0_1, b_0_1, ln0_g, ln0_b,
           W_1_0, b_1_0, W_1_1, b_1_1, ln1_g, ln1_b):
    x = jnp.pad(init_x.astype(F32), ((0, NPAD - N), (0, 0)))

    def prep(ei, ew):
        src = ei[0].astype(jnp.int32)
        dst = ei[1].astype(jnp.int32)
        ew = ew.astype(F32)
        pe = EPAD - E
        srcp = jnp.concatenate([src, jnp.zeros((pe,), jnp.int32)])
        dstp = jnp.concatenate([dst, jnp.zeros((pe,), jnp.int32)])
        ewp = jnp.concatenate([ew, jnp.zeros((pe,), F32)])
        s4 = srcp.reshape(NSUB, NGRP, GB, KE)
        d4 = dstp.reshape(NSUB, NGRP, GB, KE)
        w4 = lax.bitcast_convert_type(ewp, jnp.int32).reshape(
            NSUB, NGRP, GB, KE)
        edata = jnp.concatenate([s4, d4, w4], axis=2)
        return dstp, ewp, edata

    dst0p, ew0p, edata0 = prep(edge_index_rel0, edge_weight_rel0)
    dst1p, ew1p, edata1 = prep(edge_index_rel1, edge_weight_rel1)

    p0, p1 = _deg_parts(dst0p, ew0p, dst1p, ew1p)
    dinv0, dinv1 = _dinv(p0, p1)

    layers = (
        (W_0_0, b_0_0, W_0_1, b_0_1, ln0_g, ln0_b),
        (W_1_0, b_1_0, W_1_1, b_1_1, ln1_g, ln1_b),
    )
    last = x
    for Wa, ba, Wb, bcur, g, lb in layers:
        h0l, h0h, h1l, h1h = _matmul2(last, Wa, Wb, dinv0, dinv1)
        a0l, a0h = _agg_rel(edata0, h0l, h0h)
        a1l, a1h = _agg_rel(edata1, h1l, h1h)
        last = _combine(a0l, a0h, a1l, a1h, h0l, h0h, h1l, h1h, dinv0, dinv1,
                        ba.reshape(1, D), bcur.reshape(1, D),
                        g.reshape(1, D), lb.reshape(1, D))
    return last[:N]


# speedup vs baseline: 10.0950x; 1.3752x over previous
import dataclasses
import functools
import math

import jax
import jax.numpy as jnp
from jax import lax
from jax.experimental import pallas as pl
from jax.experimental.pallas import tpu as pltpu
from jax.experimental.pallas import tpu_sc as plsc

N = 10000
D = 256
DH = 128
E = 160000
L = 16
NSC = 2
NSUB = 16
NW = NSC * NSUB
NPAD = 10240
RPS = NPAD // NSUB
EPAD = 163840
EPT = EPAD // NSUB
KE = 64
NBLK = EPT // KE
NR = 4
GB = 8
NGRP = NBLK // GB
EA = EPAD // NW
EPS = 1e-12
F32 = jnp.float32

_mesh = plsc.VectorSubcoreMesh(core_axis_name="core", subcore_axis_name="subcore")

_sc_params = pltpu.CompilerParams()
if "needs_layout_passes" in pltpu.CompilerParams.__dataclass_fields__:
    _sc_params = dataclasses.replace(_sc_params, needs_layout_passes=False)
_sc_params_notile = dataclasses.replace(_sc_params, use_tc_tiling_on_sc=False)


def _deg_parts(dst0, ew0, dst1, ew1):
    @functools.partial(
        pl.kernel,
        out_type=(
            jax.ShapeDtypeStruct((NW, NPAD), F32),
            jax.ShapeDtypeStruct((NW, NPAD), F32),
        ),
        mesh=_mesh,
        scratch_types=[
            pltpu.VMEM((EA,), jnp.int32),
            pltpu.VMEM((EA,), F32),
            pltpu.VMEM((NPAD,), F32),
        ],
        compiler_params=_sc_params,
    )
    def kern(dst0_h, ew0_h, dst1_h, ew1_h, p0_h, p1_h, dstv, ewv, degv):
        c = lax.axis_index("core")
        s = lax.axis_index("subcore")
        w = c * NSUB + s
        for dh, eh, ph in ((dst0_h, ew0_h, p0_h), (dst1_h, ew1_h, p1_h)):
            base = w * EA
            pltpu.sync_copy(dh.at[pl.ds(base, EA)], dstv)
            pltpu.sync_copy(eh.at[pl.ds(base, EA)], ewv)

            @pl.loop(0, NPAD // L)
            def _zero(i):
                degv[pl.ds(i * L, L)] = jnp.zeros((L,), F32)

            lane = lax.iota(jnp.int32, L)

            @pl.loop(0, EA // L)
            def _acc(gi):
                dvec = dstv[pl.ds(gi * L, L)]
                evec = ewv[pl.ds(gi * L, L)]
                for k in range(L):
                    plsc.addupdate_scatter(degv, [dvec], evec, mask=lane == k)

            pltpu.sync_copy(degv, ph.at[w])

    return kern(dst0, ew0, dst1, ew1)


def _agg_rel(edata, hlo, hhi):
    @functools.partial(
        pl.kernel,
        out_type=(
            jax.ShapeDtypeStruct((NPAD, DH), F32),
            jax.ShapeDtypeStruct((NPAD, DH), F32),
        ),
        mesh=_mesh,
        scratch_types=[
            pltpu.VMEM_SHARED((NPAD, DH), F32),
            pltpu.VMEM((2, 3 * GB, KE), jnp.int32),
        ]
        + [pltpu.VMEM((KE, DH // 2), jnp.int32)] * NR
        + [pltpu.VMEM((KE, DH), F32)] * 2
        + [pltpu.SemaphoreType.DMA] * (NR + 2 + 2),
        compiler_params=_sc_params_notile,
    )
    def kern(ed_h, hlo_h, hhi_h, outlo, outhi, acc, edat, *bufsem):
        c = lax.axis_index("core")
        s = lax.axis_index("subcore")
        rows = bufsem[:NR]
        fbufs = bufsem[NR:NR + 2]
        gsem = bufsem[NR + 2:2 * NR + 2]
        ssem = bufsem[2 * NR + 2:2 * NR + 4]
        esem = bufsem[2 * NR + 4:]

        def run(h_t, out_t):
            def grp_issue(q, slot):
                pltpu.async_copy(ed_h.at[s, q], edat.at[slot], esem[slot])

            def grp_wait(q, slot):
                pltpu.make_async_copy(ed_h.at[s, q], edat.at[slot],
                                      esem[slot]).wait()

            r0 = fbufs[0]

            @pl.loop(0, KE)
            def _zb(r):
                for ch in range(DH // L):
                    r0[r, pl.ds(ch * L, L)] = jnp.zeros((L,), F32)

            @pl.loop(0, RPS // KE)
            def _za(j):
                pltpu.sync_copy(r0, acc.at[pl.ds(s * RPS + j * KE, KE)])

            plsc.subcore_barrier()

            def scale(slot, b, rbuf, fb):
                @plsc.parallel_loop(0, KE, unroll=4)
                def _sc(e):
                    eidx = jnp.full((L,), 0, jnp.int32) + e
                    svec = jnp.full((L,), slot, jnp.int32)
                    rvec = jnp.full((L,), 2 * GB + b, jnp.int32)
                    wv = plsc.bitcast(
                        plsc.load_gather(edat, [svec, rvec, eidx]), F32)
                    for ch in range(DH // (2 * L)):
                        iv = rbuf[e, pl.ds(ch * L, L)]
                        bv = plsc.bitcast(iv, jnp.bfloat16)
                        f0, f1 = plsc.unpack(
                            bv, format=plsc.PackFormat.INTERLEAVED,
                            preferred_element_type=F32)
                        fb[e, pl.ds(2 * ch * L, L)] = f0 * wv
                        fb[e, pl.ds((2 * ch + 1) * L, L)] = f1 * wv

            grp_issue(0, 0)
            grp_wait(0, 0)
            pltpu.async_copy(h_t.at[edat.at[0, 0]], rows[0], gsem[0])
            pltpu.async_copy(h_t.at[edat.at[0, 1]], rows[1], gsem[1])

            @pl.loop(0, NBLK, step=2 * GB)
            def _outer(go):
                qbase = go // GB
                for gb in range(2 * GB):
                    half, b = divmod(gb, GB)
                    g = go + gb
                    q1 = qbase + half + 1
                    br = gb % NR
                    b2 = (gb + 2) % NR
                    gb2 = gb + 2
                    slot2, b2r = divmod(gb2, GB)
                    slot2 %= 2

                    fb = fbufs[gb % 2]

                    if b == 2:
                        @pl.when(q1 < NGRP)
                        def _gi():
                            grp_issue(q1, 1 - half)
                    if b == GB - 2:
                        @pl.when(q1 < NGRP)
                        def _gw():
                            grp_wait(q1, 1 - half)

                    @pl.when(g >= 2)
                    def _wsc():
                        pltpu.make_async_copy(
                            fb, acc.at[edat.at[half, GB + b]],
                            ssem[gb % 2]).wait()

                    pltpu.make_async_copy(h_t.at[edat.at[half, b]],
                                          rows[br], gsem[br]).wait()
                    scale(half, b, rows[br], fb)
                    pltpu.async_copy(fb, acc.at[edat.at[half, GB + b]],
                                     ssem[gb % 2], add=True)

                    @pl.when(g + 2 < NBLK)
                    def _pref():
                        pltpu.async_copy(h_t.at[edat.at[slot2, b2r]],
                                         rows[b2], gsem[b2])

            for j in range(2):
                g = NBLK - 2 + j
                pltpu.make_async_copy(fbufs[g % 2],
                                      acc.at[edat.at[1, GB + (g % GB)]],
                                      ssem[g % 2]).wait()

            plsc.subcore_barrier()
            pltpu.sync_copy(acc.at[pl.ds(s * RPS, RPS)],
                            out_t.at[pl.ds(s * RPS, RPS)])

        @pl.when(c == 0)
        def _c0():
            run(hlo_h, outlo)

        @pl.when(c == 1)
        def _c1():
            run(hhi_h, outhi)

    return kern(edata, hlo, hhi)


def _dinv(p0, p1):
    CB = 1024

    def body(p0_ref, p1_ref, o0, o1):
        ones = jnp.ones((NW, 1), F32)
        dn = (((0,), (0,)), ((), ()))
        for p_ref, o in ((p0_ref, o0), (p1_ref, o1)):
            deg = lax.dot_general(p_ref[...], ones, dn,
                                  precision=lax.Precision.HIGHEST)
            o[...] = lax.rsqrt(deg + 1.0)

    return pl.pallas_call(
        body,
        grid=(NPAD // CB,),
        in_specs=[pl.BlockSpec((NW, CB), lambda i: (0, i))] * 2,
        out_specs=[pl.BlockSpec((CB, 1), lambda i: (i, 0))] * 2,
        out_shape=[jax.ShapeDtypeStruct((NPAD, 1), F32)] * 2,
    )(p0, p1)


def _matmul2(x, w0, w1, dinv0, dinv1):
    RB = 512

    def body(x_ref, w0_ref, w1_ref, d0_ref, d1_ref, o0l, o0h, o1l, o1h):
        xb = x_ref[...]
        for w_ref, d_ref, ol, oh in ((w0_ref, d0_ref, o0l, o0h),
                                     (w1_ref, d1_ref, o1l, o1h)):
            h = (jnp.dot(xb, w_ref[...]) * d_ref[...]).astype(jnp.bfloat16)
            ol[...] = h[:, :DH]
            oh[...] = h[:, DH:]

    return pl.pallas_call(
        body,
        grid=(NPAD // RB,),
        in_specs=[
            pl.BlockSpec((RB, D), lambda i: (i, 0)),
            pl.BlockSpec((D, D), lambda i: (0, 0)),
            pl.BlockSpec((D, D), lambda i: (0, 0)),
            pl.BlockSpec((RB, 1), lambda i: (i, 0)),
            pl.BlockSpec((RB, 1), lambda i: (i, 0)),
        ],
        out_specs=[pl.BlockSpec((RB, DH), lambda i: (i, 0))] * 4,
        out_shape=[jax.ShapeDtypeStruct((NPAD, DH), jnp.bfloat16)] * 4,
    )(x, w0, w1, dinv0, dinv1)


def _combine(a0l, a0h, a1l, a1h, h0l, h0h, h1l, h1h, dinv0, dinv1,
             b0, b1, g, bb):
    RB = 512
    inv_sqrt2 = 1.0 / math.sqrt(2.0)

    def body(a0l_r, a0h_r, a1l_r, a1h_r, h0l_r, h0h_r, h1l_r, h1h_r,
             d0_r, d1_r, b0_r, b1_r, g_r, bb_r, o_r):
        d0 = d0_r[...]
        d1 = d1_r[...]
        lo = (d0 * (a0l_r[...] + h0l_r[...].astype(F32))
              + d1 * (a1l_r[...] + h1l_r[...].astype(F32)))
        hi = (d0 * (a0h_r[...] + h0h_r[...].astype(F32))
              + d1 * (a1h_r[...] + h1h_r[...].astype(F32)))
        x = jnp.concatenate([lo, hi], axis=1) + b0_r[...] + b1_r[...]
        mu = jnp.mean(x, axis=1, keepdims=True)
        xc = x - mu
        var = jnp.mean(xc * xc, axis=1, keepdims=True)
        x = xc * lax.rsqrt(var + EPS) * g_r[...] + bb_r[...]
        o_r[...] = x * 0.5 * (1.0 + lax.erf(x * inv_sqrt2))

    row = lambda v: pl.BlockSpec((RB, DH), lambda i: (i, 0))
    return pl.pallas_call(
        body,
        grid=(NPAD // RB,),
        in_specs=(
            [pl.BlockSpec((RB, DH), lambda i: (i, 0))] * 8
            + [pl.BlockSpec((RB, 1), lambda i: (i, 0))] * 2
            + [pl.BlockSpec((1, D), lambda i: (0, 0))] * 4
        ),
        out_specs=pl.BlockSpec((RB, D), lambda i: (i, 0)),
        out_shape=jax.ShapeDtypeStruct((NPAD, D), F32),
    )(a0l, a0h, a1l, a1h, h0l, h0h, h1l, h1h, dinv0, dinv1, b0, b1, g, bb)


def kernel(init_x, edge_index_rel0, edge_weight_rel0, edge_index_rel1,
           edge_weight_rel1, W_0_0, b_0_0, W_0_1, b_0_1, ln0_g, ln0_b,
           W_1_0, b_1_0, W_1_1, b_1_1, ln1_g, ln1_b):
    x = jnp.pad(init_x.astype(F32), ((0, NPAD - N), (0, 0)))

    def prep(ei, ew):
        src = ei[0].astype(jnp.int32)
        dst = ei[1].astype(jnp.int32)
        ew = ew.astype(F32)
        pe = EPAD - E
        srcp = jnp.concatenate([src, jnp.zeros((pe,), jnp.int32)])
        dstp = jnp.concatenate([dst, jnp.zeros((pe,), jnp.int32)])
        ewp = jnp.concatenate([ew, jnp.zeros((pe,), F32)])
        s4 = srcp.reshape(NSUB, NGRP, GB, KE)
        d4 = dstp.reshape(NSUB, NGRP, GB, KE)
        w4 = lax.bitcast_convert_type(ewp, jnp.int32).reshape(
            NSUB, NGRP, GB, KE)
        edata = jnp.concatenate([s4, d4, w4], axis=2)
        return dstp, ewp, edata

    dst0p, ew0p, edata0 = prep(edge_index_rel0, edge_weight_rel0)
    dst1p, ew1p, edata1 = prep(edge_index_rel1, edge_weight_rel1)

    p0, p1 = _deg_parts(dst0p, ew0p, dst1p, ew1p)
    dinv0, dinv1 = _dinv(p0, p1)

    layers = (
        (W_0_0, b_0_0, W_0_1, b_0_1, ln0_g, ln0_b),
        (W_1_0, b_1_0, W_1_1, b_1_1, ln1_g, ln1_b),
    )
    def packtab(hb):
        t = hb.reshape(NPAD, 4, 2, L).transpose(0, 1, 3, 2)
        return lax.bitcast_convert_type(t, jnp.int32).reshape(NPAD, DH // 2)

    last = x
    for Wa, ba, Wb, bcur, g, lb in layers:
        h0l, h0h, h1l, h1h = _matmul2(last, Wa, Wb, dinv0, dinv1)
        a0l, a0h = _agg_rel(edata0, packtab(h0l), packtab(h0h))
        a1l, a1h = _agg_rel(edata1, packtab(h1l), packtab(h1h))
        last = _combine(a0l, a0h, a1l, a1h, h0l, h0h, h1l, h1h, dinv0, dinv1,
                        ba.reshape(1, D), bcur.reshape(1, D),
                        g.reshape(1, D), lb.reshape(1, D))
    return last[:N]


# speedup vs baseline: 10.2591x; 1.0163x over previous
import dataclasses
import functools
import math

import jax
import jax.numpy as jnp
from jax import lax
from jax.experimental import pallas as pl
from jax.experimental.pallas import tpu as pltpu
from jax.experimental.pallas import tpu_sc as plsc

N = 10000
D = 256
DH = 128
E = 160000
L = 16
NSC = 2
NSUB = 16
NW = NSC * NSUB
NPAD = 10240
RPS = NPAD // NSUB
EPAD = 163840
EPT = EPAD // NSUB
KE = 64
NBLK = EPT // KE
NR = 4
GB = 8
NGRP = NBLK // GB
EA = EPAD // NW
EPS = 1e-12
F32 = jnp.float32

_mesh = plsc.VectorSubcoreMesh(core_axis_name="core", subcore_axis_name="subcore")

_sc_params = pltpu.CompilerParams()
if "needs_layout_passes" in pltpu.CompilerParams.__dataclass_fields__:
    _sc_params = dataclasses.replace(_sc_params, needs_layout_passes=False)
_sc_params_notile = dataclasses.replace(_sc_params, use_tc_tiling_on_sc=False)


def _deg_parts(dst0, ew0, dst1, ew1):
    @functools.partial(
        pl.kernel,
        out_type=(
            jax.ShapeDtypeStruct((NW, NPAD), F32),
            jax.ShapeDtypeStruct((NW, NPAD), F32),
        ),
        mesh=_mesh,
        scratch_types=[
            pltpu.VMEM((EA,), jnp.int32),
            pltpu.VMEM((EA,), F32),
            pltpu.VMEM((NPAD,), F32),
        ],
        compiler_params=_sc_params,
    )
    def kern(dst0_h, ew0_h, dst1_h, ew1_h, p0_h, p1_h, dstv, ewv, degv):
        c = lax.axis_index("core")
        s = lax.axis_index("subcore")
        w = c * NSUB + s
        for dh, eh, ph in ((dst0_h, ew0_h, p0_h), (dst1_h, ew1_h, p1_h)):
            base = w * EA
            pltpu.sync_copy(dh.at[pl.ds(base, EA)], dstv)
            pltpu.sync_copy(eh.at[pl.ds(base, EA)], ewv)

            @pl.loop(0, NPAD // L)
            def _zero(i):
                degv[pl.ds(i * L, L)] = jnp.zeros((L,), F32)

            lane = lax.iota(jnp.int32, L)

            @pl.loop(0, EA // L)
            def _acc(gi):
                dvec = dstv[pl.ds(gi * L, L)]
                evec = ewv[pl.ds(gi * L, L)]
                for k in range(L):
                    plsc.addupdate_scatter(degv, [dvec], evec, mask=lane == k)

            pltpu.sync_copy(degv, ph.at[w])

    return kern(dst0, ew0, dst1, ew1)


def _agg_rel(edata, hlo, hhi):
    @functools.partial(
        pl.kernel,
        out_type=(
            jax.ShapeDtypeStruct((NPAD, DH), F32),
            jax.ShapeDtypeStruct((NPAD, DH), F32),
        ),
        mesh=_mesh,
        scratch_types=[
            pltpu.VMEM_SHARED((NPAD, DH), F32),
            pltpu.VMEM((2, 3 * GB, KE), jnp.int32),
        ]
        + [pltpu.VMEM((KE, DH // 2), jnp.int32)] * NR
        + [pltpu.VMEM((KE, DH), F32)] * 2
        + [pltpu.SemaphoreType.DMA] * (NR + 2 + 2),
        compiler_params=_sc_params_notile,
    )
    def kern(ed_h, hlo_h, hhi_h, outlo, outhi, acc, edat, *bufsem):
        c = lax.axis_index("core")
        s = lax.axis_index("subcore")
        rows = bufsem[:NR]
        fbufs = bufsem[NR:NR + 2]
        gsem = bufsem[NR + 2:2 * NR + 2]
        ssem = bufsem[2 * NR + 2:2 * NR + 4]
        esem = bufsem[2 * NR + 4:]

        def run(h_t, out_t):
            def grp_issue(q, slot):
                pltpu.async_copy(ed_h.at[s, q], edat.at[slot], esem[slot])

            def grp_wait(q, slot):
                pltpu.make_async_copy(ed_h.at[s, q], edat.at[slot],
                                      esem[slot]).wait()

            r0 = fbufs[0]

            @pl.loop(0, KE)
            def _zb(r):
                for ch in range(DH // L):
                    r0[r, pl.ds(ch * L, L)] = jnp.zeros((L,), F32)

            @pl.loop(0, RPS // KE)
            def _za(j):
                pltpu.sync_copy(r0, acc.at[pl.ds(s * RPS + j * KE, KE)])

            plsc.subcore_barrier()

            def scale(slot, b, rbuf, fb):
                @plsc.parallel_loop(0, KE, unroll=4)
                def _sc(e):
                    eidx = jnp.full((L,), 0, jnp.int32) + e
                    svec = jnp.full((L,), slot, jnp.int32)
                    rvec = jnp.full((L,), 2 * GB + b, jnp.int32)
                    wv = plsc.bitcast(
                        plsc.load_gather(edat, [svec, rvec, eidx]), F32)
                    for ch in range(DH // (2 * L)):
                        iv = rbuf[e, pl.ds(ch * L, L)]
                        bv = plsc.bitcast(iv, jnp.bfloat16)
                        f0, f1 = plsc.unpack(
                            bv, format=plsc.PackFormat.INTERLEAVED,
                            preferred_element_type=F32)
                        fb[e, pl.ds(2 * ch * L, L)] = f0 * wv
                        fb[e, pl.ds((2 * ch + 1) * L, L)] = f1 * wv

            grp_issue(0, 0)
            grp_wait(0, 0)
            pltpu.async_copy(h_t.at[edat.at[0, 0]], rows[0], gsem[0])
            pltpu.async_copy(h_t.at[edat.at[0, 1]], rows[1], gsem[1])
            pltpu.async_copy(h_t.at[edat.at[0, 2]], rows[2], gsem[2])

            @pl.loop(0, NBLK, step=2 * GB)
            def _outer(go):
                qbase = go // GB
                for gb in range(2 * GB):
                    half, b = divmod(gb, GB)
                    g = go + gb
                    q1 = qbase + half + 1
                    br = gb % NR
                    b3 = (gb + 3) % NR
                    slot3, b3r = divmod(gb + 3, GB)
                    slot3 %= 2

                    fb = fbufs[gb % 2]

                    if b == 2:
                        @pl.when(q1 < NGRP)
                        def _gi():
                            grp_issue(q1, 1 - half)
                    if b == GB - 3:
                        @pl.when(q1 < NGRP)
                        def _gw():
                            grp_wait(q1, 1 - half)

                    @pl.when(g >= 2)
                    def _wsc():
                        pltpu.make_async_copy(
                            fb, acc.at[edat.at[half, GB + b]],
                            ssem[gb % 2]).wait()

                    pltpu.make_async_copy(h_t.at[edat.at[half, b]],
                                          rows[br], gsem[br]).wait()
                    scale(half, b, rows[br], fb)
                    pltpu.async_copy(fb, acc.at[edat.at[half, GB + b]],
                                     ssem[gb % 2], add=True)

                    @pl.when(g + 3 < NBLK)
                    def _pref():
                        pltpu.async_copy(h_t.at[edat.at[slot3, b3r]],
                                         rows[b3], gsem[b3])

            for j in range(2):
                g = NBLK - 2 + j
                pltpu.make_async_copy(fbufs[g % 2],
                                      acc.at[edat.at[1, GB + (g % GB)]],
                                      ssem[g % 2]).wait()

            plsc.subcore_barrier()
            pltpu.sync_copy(acc.at[pl.ds(s * RPS, RPS)],
                            out_t.at[pl.ds(s * RPS, RPS)])

        @pl.when(c == 0)
        def _c0():
            run(hlo_h, outlo)

        @pl.when(c == 1)
        def _c1():
            run(hhi_h, outhi)

    return kern(edata, hlo, hhi)


def _dinv(p0, p1):
    CB = 1024

    def body(p0_ref, p1_ref, o0, o1):
        ones = jnp.ones((NW, 1), F32)
        dn = (((0,), (0,)), ((), ()))
        for p_ref, o in ((p0_ref, o0), (p1_ref, o1)):
            deg = lax.dot_general(p_ref[...], ones, dn,
                                  precision=lax.Precision.HIGHEST)
            o[...] = lax.rsqrt(deg + 1.0)

    return pl.pallas_call(
        body,
        grid=(NPAD // CB,),
        in_specs=[pl.BlockSpec((NW, CB), lambda i: (0, i))] * 2,
        out_specs=[pl.BlockSpec((CB, 1), lambda i: (i, 0))] * 2,
        out_shape=[jax.ShapeDtypeStruct((NPAD, 1), F32)] * 2,
    )(p0, p1)


def _matmul2(x, w0, w1, dinv0, dinv1):
    RB = 512

    def body(x_ref, w0_ref, w1_ref, d0_ref, d1_ref, o0l, o0h, o1l, o1h):
        xb = x_ref[...]
        for w_ref, d_ref, ol, oh in ((w0_ref, d0_ref, o0l, o0h),
                                     (w1_ref, d1_ref, o1l, o1h)):
            h = (jnp.dot(xb, w_ref[...]) * d_ref[...]).astype(jnp.bfloat16)
            ol[...] = h[:, :DH]
            oh[...] = h[:, DH:]

    return pl.pallas_call(
        body,
        grid=(NPAD // RB,),
        in_specs=[
            pl.BlockSpec((RB, D), lambda i: (i, 0)),
            pl.BlockSpec((D, D), lambda i: (0, 0)),
            pl.BlockSpec((D, D), lambda i: (0, 0)),
            pl.BlockSpec((RB, 1), lambda i: (i, 0)),
            pl.BlockSpec((RB, 1), lambda i: (i, 0)),
        ],
        out_specs=[pl.BlockSpec((RB, DH), lambda i: (i, 0))] * 4,
        out_shape=[jax.ShapeDtypeStruct((NPAD, DH), jnp.bfloat16)] * 4,
    )(x, w0, w1, dinv0, dinv1)


def _combine(a0l, a0h, a1l, a1h, h0l, h0h, h1l, h1h, dinv0, dinv1,
             b0, b1, g, bb):
    RB = 512
    inv_sqrt2 = 1.0 / math.sqrt(2.0)

    def body(a0l_r, a0h_r, a1l_r, a1h_r, h0l_r, h0h_r, h1l_r, h1h_r,
             d0_r, d1_r, b0_r, b1_r, g_r, bb_r, o_r):
        d0 = d0_r[...]
        d1 = d1_r[...]
        lo = (d0 * (a0l_r[...] + h0l_r[...].astype(F32))
              + d1 * (a1l_r[...] + h1l_r[...].astype(F32)))
        hi = (d0 * (a0h_r[...] + h0h_r[...].astype(F32))
              + d1 * (a1h_r[...] + h1h_r[...].astype(F32)))
        x = jnp.concatenate([lo, hi], axis=1) + b0_r[...] + b1_r[...]
        mu = jnp.mean(x, axis=1, keepdims=True)
        xc = x - mu
        var = jnp.mean(xc * xc, axis=1, keepdims=True)
        x = xc * lax.rsqrt(var + EPS) * g_r[...] + bb_r[...]
        o_r[...] = x * 0.5 * (1.0 + lax.erf(x * inv_sqrt2))

    row = lambda v: pl.BlockSpec((RB, DH), lambda i: (i, 0))
    return pl.pallas_call(
        body,
        grid=(NPAD // RB,),
        in_specs=(
            [pl.BlockSpec((RB, DH), lambda i: (i, 0))] * 8
            + [pl.BlockSpec((RB, 1), lambda i: (i, 0))] * 2
            + [pl.BlockSpec((1, D), lambda i: (0, 0))] * 4
        ),
        out_specs=pl.BlockSpec((RB, D), lambda i: (i, 0)),
        out_shape=jax.ShapeDtypeStruct((NPAD, D), F32),
    )(a0l, a0h, a1l, a1h, h0l, h0h, h1l, h1h, dinv0, dinv1, b0, b1, g, bb)


def kernel(init_x, edge_index_rel0, edge_weight_rel0, edge_index_rel1,
           edge_weight_rel1, W_0_0, b_0_0, W_0_1, b_0_1, ln0_g, ln0_b,
           W_1_0, b_1_0, W_1_1, b_1_1, ln1_g, ln1_b):
    x = jnp.pad(init_x.astype(F32), ((0, NPAD - N), (0, 0)))

    def prep(ei, ew):
        src = ei[0].astype(jnp.int32)
        dst = ei[1].astype(jnp.int32)
        ew = ew.astype(F32)
        pe = EPAD - E
        srcp = jnp.concatenate([src, jnp.zeros((pe,), jnp.int32)])
        dstp = jnp.concatenate([dst, jnp.zeros((pe,), jnp.int32)])
        ewp = jnp.concatenate([ew, jnp.zeros((pe,), F32)])
        s4 = srcp.reshape(NSUB, NGRP, GB, KE)
        d4 = dstp.reshape(NSUB, NGRP, GB, KE)
        w4 = lax.bitcast_convert_type(ewp, jnp.int32).reshape(
            NSUB, NGRP, GB, KE)
        edata = jnp.concatenate([s4, d4, w4], axis=2)
        return dstp, ewp, edata

    dst0p, ew0p, edata0 = prep(edge_index_rel0, edge_weight_rel0)
    dst1p, ew1p, edata1 = prep(edge_index_rel1, edge_weight_rel1)

    p0, p1 = _deg_parts(dst0p, ew0p, dst1p, ew1p)
    dinv0, dinv1 = _dinv(p0, p1)

    layers = (
        (W_0_0, b_0_0, W_0_1, b_0_1, ln0_g, ln0_b),
        (W_1_0, b_1_0, W_1_1, b_1_1, ln1_g, ln1_b),
    )
    def packtab(hb):
        t = hb.reshape(NPAD, 4, 2, L).transpose(0, 1, 3, 2)
        return lax.bitcast_convert_type(t, jnp.int32).reshape(NPAD, DH // 2)

    last = x
    for Wa, ba, Wb, bcur, g, lb in layers:
        h0l, h0h, h1l, h1h = _matmul2(last, Wa, Wb, dinv0, dinv1)
        a0l, a0h = _agg_rel(edata0, packtab(h0l), packtab(h0h))
        a1l, a1h = _agg_rel(edata1, packtab(h1l), packtab(h1h))
        last = _combine(a0l, a0h, a1l, a1h, h0l, h0h, h1l, h1h, dinv0, dinv1,
                        ba.reshape(1, D), bcur.reshape(1, D),
                        g.reshape(1, D), lb.reshape(1, D))
    return last[:N]
